# baseline (speedup 1.0000x reference)
import numpy as np

import jax
import jax.numpy as jnp
from jax import lax
from jax.experimental import pallas as pl
from jax.experimental.pallas import tpu as pltpu


def _build_schedule(nb):
    rows = []
    for r in range(nb):
        js = [j for j in range(nb) if j != r] + [r]
        for j in js:
            diag = 1 if j == r else 0
            fuse = 1 if j < r else 0
            wr = 1 if (diag and r == nb - 1) else 0
            rows.append((r, j, 1, fuse, diag, wr, nb - 1))
    for r in range(nb - 1):
        for j in range(r + 1, nb):
            wr = 1 if j == nb - 1 else 0
            rows.append((r, j, 0, 1, 0, wr, r))
    return np.asarray(rows, dtype=np.int32).T.copy()


def _make_s1_kernel(n, nhid):
    def body(x_ref, w1_ref, o_ref):
        o_ref[:n, :nhid] = jnp.dot(x_ref[...], w1_ref[...],
                                   preferred_element_type=jnp.float32)
        if o_ref.shape[0] > n:
            o_ref[n:, :nhid] = jnp.zeros_like(o_ref[n:, :nhid])
        o_ref[:, nhid:] = jnp.zeros_like(o_ref[:, nhid:])
    return body


def _make_main_kernel(nb, b, n, nhid, nclass):
    vc = n - (nb - 1) * b

    def body(s_ref, sh_ref, adj_ref, b1_ref, w2_ref, b2_ref, o_ref,
             p_ref, oacc_ref):
        t = pl.program_id(0)
        ri = s_ref[0, t]
        ci = s_ref[1, t]
        pa = s_ref[2, t]
        fuse = s_ref[3, t]
        diag = s_ref[4, t]
        wr = s_ref[5, t]

        @pl.when(t == 0)
        def _():
            p_ref[...] = jnp.zeros_like(p_ref)
            oacc_ref[...] = jnp.zeros_like(oacc_ref)

        if vc < b:
            @pl.when(ci == nb - 1)
            def _():
                adj_ref[:, vc:] = jnp.zeros_like(adj_ref[:, vc:])

        blk = adj_ref[...]
        acc = jnp.dot(blk, sh_ref[pl.ds(ci * b, b), :],
                      preferred_element_type=jnp.float32)

        @pl.when(pa == 1)
        def _():
            p_ref[...] += acc[:, :nhid]

        @pl.when(fuse == 1)
        def _():
            oacc_ref[pl.ds(ri * b, b), :] += acc[:, nhid:]

        @pl.when(diag == 1)
        def _():
            hid = jnp.maximum(p_ref[...] + b1_ref[...], 0.0)
            hrow = jnp.dot(hid, w2_ref[...],
                           preferred_element_type=jnp.float32)
            if vc < b:
                vr = jnp.minimum(b, n - ri * b)
                riota = lax.broadcasted_iota(jnp.int32, (b, nclass), 0)
                hrow = jnp.where(riota < vr, hrow, 0.0)
            sh_ref[pl.ds(ri * b, b), nhid:] = hrow
            oacc_ref[pl.ds(ri * b, b), :] += jnp.dot(
                blk, hrow, preferred_element_type=jnp.float32)
            p_ref[...] = jnp.zeros_like(p_ref)

        @pl.when(wr == 1)
        def _():
            o = oacc_ref[pl.ds(ri * b, b), :] + b2_ref[...]
            m = jnp.max(o, axis=1, keepdims=True)
            lse = jnp.log(jnp.sum(jnp.exp(o - m), axis=1, keepdims=True))
            o_ref[...] = o - m - lse

    return body


def kernel(x, adj, W1, b1, W2, b2):
    n, nfeat = x.shape
    nhid = W1.shape[1]
    nclass = W2.shape[1]
    b1r = b1.reshape(1, nhid)
    b2r = b2.reshape(1, nclass)

    B = 1920 if n > 1920 else -(-(n // 5) // 128) * 128
    nb = -(-n // B)
    npad = nb * B
    sched = jnp.asarray(_build_schedule(nb))
    T = sched.shape[1]

    sh = pl.pallas_call(
        _make_s1_kernel(n, nhid),
        out_shape=jax.ShapeDtypeStruct((npad, nhid + nclass), jnp.float32),
    )(x, W1)

    grid_spec = pltpu.PrefetchScalarGridSpec(
        num_scalar_prefetch=1,
        grid=(T,),
        in_specs=[
            pl.BlockSpec((npad, nhid + nclass), lambda t, s: (0, 0)),
            pl.BlockSpec((B, B), lambda t, s: (s[0, t], s[1, t])),
            pl.BlockSpec((1, nhid), lambda t, s: (0, 0)),
            pl.BlockSpec((nhid, nclass), lambda t, s: (0, 0)),
            pl.BlockSpec((1, nclass), lambda t, s: (0, 0)),
        ],
        out_specs=pl.BlockSpec((B, nclass), lambda t, s: (s[6, t], 0)),
        scratch_shapes=[
            pltpu.VMEM((B, nhid), jnp.float32),
            pltpu.VMEM((npad, nclass), jnp.float32),
        ],
    )

    return pl.pallas_call(
        _make_main_kernel(nb, B, n, nhid, nclass),
        grid_spec=grid_spec,
        out_shape=jax.ShapeDtypeStruct((n, nclass), jnp.float32),
        compiler_params=pltpu.CompilerParams(
            vmem_limit_bytes=63 * 1024 * 1024),
    )(sched, sh, adj, b1r, W2, b2r)


# speedup vs baseline: 1.1256x; 1.1256x over previous
import numpy as np

import jax
import jax.numpy as jnp
from jax import lax
from jax.experimental import pallas as pl
from jax.experimental.pallas import tpu as pltpu


def _build_schedule(nb):
    rows = []
    for r in range(nb):
        js = [j for j in range(nb) if j != r] + [r]
        for j in js:
            diag = 1 if j == r else 0
            fuse = 1 if j < r else 0
            wr = 1 if (diag and r == nb - 1) else 0
            rows.append((r, j, 1, fuse, diag, wr, nb - 1))
    for r in range(nb - 1):
        for j in range(r + 1, nb):
            wr = 1 if j == nb - 1 else 0
            rows.append((r, j, 0, 1, 0, wr, r))
    return np.asarray(rows, dtype=np.int32).T.copy()


def _make_s1_kernel(n, nhid):
    def body(x_ref, w1_ref, o_ref):
        o_ref[:n, :nhid] = jnp.dot(x_ref[...], w1_ref[...],
                                   preferred_element_type=jnp.float32)
        if o_ref.shape[0] > n:
            o_ref[n:, :nhid] = jnp.zeros_like(o_ref[n:, :nhid])
        o_ref[:, nhid:] = jnp.zeros_like(o_ref[:, nhid:])
    return body


def _make_main_kernel(nb, b, n, nhid, nclass):
    vc = n - (nb - 1) * b

    def body(s_ref, sh_ref, adj_ref, b1_ref, w2_ref, b2_ref, o_ref,
             p_ref, oacc_ref):
        t = pl.program_id(0)
        ri = s_ref[0, t]
        ci = s_ref[1, t]
        pa = s_ref[2, t]
        fuse = s_ref[3, t]
        diag = s_ref[4, t]
        wr = s_ref[5, t]

        @pl.when(t == 0)
        def _():
            p_ref[...] = jnp.zeros_like(p_ref)
            oacc_ref[...] = jnp.zeros_like(oacc_ref)

        if vc < b:
            @pl.when(ci == nb - 1)
            def _():
                adj_ref[:, vc:] = jnp.zeros_like(adj_ref[:, vc:])

        blk = adj_ref[...]
        acc = jnp.dot(blk, sh_ref[pl.ds(ci * b, b), :],
                      preferred_element_type=jnp.float32)

        @pl.when(pa == 1)
        def _():
            p_ref[...] += acc[:, :nhid]

        @pl.when(fuse == 1)
        def _():
            oacc_ref[pl.ds(ri * b, b), :] += acc[:, nhid:]

        @pl.when(diag == 1)
        def _():
            hid = jnp.maximum(p_ref[...] + b1_ref[...], 0.0)
            hrow = jnp.dot(hid, w2_ref[...],
                           preferred_element_type=jnp.float32)
            if vc < b:
                vr = jnp.minimum(b, n - ri * b)
                riota = lax.broadcasted_iota(jnp.int32, (b, nclass), 0)
                hrow = jnp.where(riota < vr, hrow, 0.0)
            sh_ref[pl.ds(ri * b, b), nhid:] = hrow
            oacc_ref[pl.ds(ri * b, b), :] += jnp.dot(
                blk, hrow, preferred_element_type=jnp.float32)
            p_ref[...] = jnp.zeros_like(p_ref)

        @pl.when(wr == 1)
        def _():
            o = oacc_ref[pl.ds(ri * b, b), :] + b2_ref[...]
            m = jnp.max(o, axis=1, keepdims=True)
            lse = jnp.log(jnp.sum(jnp.exp(o - m), axis=1, keepdims=True))
            o_ref[...] = o - m - lse

    return body


def kernel(x, adj, W1, b1, W2, b2):
    n, nfeat = x.shape
    nhid = W1.shape[1]
    nclass = W2.shape[1]
    b1r = b1.reshape(1, nhid)
    b2r = b2.reshape(1, nclass)

    B = 1792 if n > 1792 else -(-(n // 5) // 128) * 128
    nb = -(-n // B)
    npad = nb * B
    sched = jnp.asarray(_build_schedule(nb))
    T = sched.shape[1]

    sh = pl.pallas_call(
        _make_s1_kernel(n, nhid),
        out_shape=jax.ShapeDtypeStruct((npad, nhid + nclass), jnp.float32),
    )(x, W1)

    grid_spec = pltpu.PrefetchScalarGridSpec(
        num_scalar_prefetch=1,
        grid=(T,),
        in_specs=[
            pl.BlockSpec((npad, nhid + nclass), lambda t, s: (0, 0)),
            pl.BlockSpec((B, B), lambda t, s: (s[0, t], s[1, t])),
            pl.BlockSpec((1, nhid), lambda t, s: (0, 0)),
            pl.BlockSpec((nhid, nclass), lambda t, s: (0, 0)),
            pl.BlockSpec((1, nclass), lambda t, s: (0, 0)),
        ],
        out_specs=pl.BlockSpec((B, nclass), lambda t, s: (s[6, t], 0)),
        scratch_shapes=[
            pltpu.VMEM((B, nhid), jnp.float32),
            pltpu.VMEM((npad, nclass), jnp.float32),
        ],
    )

    return pl.pallas_call(
        _make_main_kernel(nb, B, n, nhid, nclass),
        grid_spec=grid_spec,
        out_shape=jax.ShapeDtypeStruct((n, nclass), jnp.float32),
        compiler_params=pltpu.CompilerParams(
            vmem_limit_bytes=63 * 1024 * 1024),
    )(sched, sh, adj, b1r, W2, b2r)


# speedup vs baseline: 1.1838x; 1.0517x over previous
import numpy as np

import jax
import jax.numpy as jnp
from jax import lax
from jax.experimental import pallas as pl
from jax.experimental.pallas import tpu as pltpu


def _build_schedule(nb):
    rows = []
    for r in range(nb):
        js = [j for j in range(nb) if j != r] + [r]
        for j in js:
            diag = 1 if j == r else 0
            fuse = 1 if j < r else 0
            wr = 1 if (diag and r == nb - 1) else 0
            rows.append((r, j, 1, fuse, diag, wr, nb - 1))
    for r in range(nb - 1):
        for j in range(r + 1, nb):
            wr = 1 if j == nb - 1 else 0
            rows.append((r, j, 0, 1, 0, wr, r))
    return np.asarray(rows, dtype=np.int32).T.copy()


def _make_s1_kernel(n, nhid):
    def body(x_ref, w1_ref, o_ref):
        o_ref[:n, :nhid] = jnp.dot(x_ref[...], w1_ref[...],
                                   preferred_element_type=jnp.float32)
        if o_ref.shape[0] > n:
            o_ref[n:, :nhid] = jnp.zeros_like(o_ref[n:, :nhid])
        o_ref[:, nhid:] = jnp.zeros_like(o_ref[:, nhid:])
    return body


def _make_main_kernel(nb, b, n, nhid, nclass):
    vc = n - (nb - 1) * b

    def body(s_ref, sh_ref, adj_ref, b1_ref, w2_ref, b2_ref, o_ref,
             p_ref, oacc_ref):
        t = pl.program_id(0)
        ri = s_ref[0, t]
        ci = s_ref[1, t]
        pa = s_ref[2, t]
        fuse = s_ref[3, t]
        diag = s_ref[4, t]
        wr = s_ref[5, t]

        @pl.when(t == 0)
        def _():
            p_ref[...] = jnp.zeros_like(p_ref)
            oacc_ref[...] = jnp.zeros_like(oacc_ref)

        if vc < b:
            @pl.when(ci == nb - 1)
            def _():
                adj_ref[:, vc:] = jnp.zeros_like(adj_ref[:, vc:])

        blk = adj_ref[...]
        acc = jnp.dot(blk, sh_ref[pl.ds(ci * b, b), :],
                      preferred_element_type=jnp.float32)

        @pl.when(pa == 1)
        def _():
            p_ref[...] += acc[:, :nhid]

        @pl.when(fuse == 1)
        def _():
            oacc_ref[pl.ds(ri * b, b), :] += acc[:, nhid:]

        @pl.when(diag == 1)
        def _():
            hid = jnp.maximum(p_ref[...] + b1_ref[...], 0.0)
            hrow = jnp.dot(hid, w2_ref[...],
                           preferred_element_type=jnp.float32)
            if vc < b:
                vr = jnp.minimum(b, n - ri * b)
                riota = lax.broadcasted_iota(jnp.int32, (b, nclass), 0)
                hrow = jnp.where(riota < vr, hrow, 0.0)
            sh_ref[pl.ds(ri * b, b), nhid:] = hrow
            oacc_ref[pl.ds(ri * b, b), :] += jnp.dot(
                blk, hrow, preferred_element_type=jnp.float32)
            p_ref[...] = jnp.zeros_like(p_ref)

        @pl.when(wr == 1)
        def _():
            o = oacc_ref[pl.ds(ri * b, b), :] + b2_ref[...]
            m = jnp.max(o, axis=1, keepdims=True)
            lse = jnp.log(jnp.sum(jnp.exp(o - m), axis=1, keepdims=True))
            o_ref[...] = o - m - lse

    return body


def kernel(x, adj, W1, b1, W2, b2):
    n, nfeat = x.shape
    nhid = W1.shape[1]
    nclass = W2.shape[1]
    b1r = b1.reshape(1, nhid)
    b2r = b2.reshape(1, nclass)

    B = 2048 if n > 2048 else -(-(n // 5) // 128) * 128
    nb = -(-n // B)
    npad = nb * B
    sched = jnp.asarray(_build_schedule(nb))
    T = sched.shape[1]

    sh = pl.pallas_call(
        _make_s1_kernel(n, nhid),
        out_shape=jax.ShapeDtypeStruct((npad, nhid + nclass), jnp.float32),
    )(x, W1)

    grid_spec = pltpu.PrefetchScalarGridSpec(
        num_scalar_prefetch=1,
        grid=(T,),
        in_specs=[
            pl.BlockSpec((npad, nhid + nclass), lambda t, s: (0, 0)),
            pl.BlockSpec((B, B), lambda t, s: (s[0, t], s[1, t])),
            pl.BlockSpec((1, nhid), lambda t, s: (0, 0)),
            pl.BlockSpec((nhid, nclass), lambda t, s: (0, 0)),
            pl.BlockSpec((1, nclass), lambda t, s: (0, 0)),
        ],
        out_specs=pl.BlockSpec((B, nclass), lambda t, s: (s[6, t], 0)),
        scratch_shapes=[
            pltpu.VMEM((B, nhid), jnp.float32),
            pltpu.VMEM((npad, nclass), jnp.float32),
        ],
    )

    return pl.pallas_call(
        _make_main_kernel(nb, B, n, nhid, nclass),
        grid_spec=grid_spec,
        out_shape=jax.ShapeDtypeStruct((n, nclass), jnp.float32),
        compiler_params=pltpu.CompilerParams(
            vmem_limit_bytes=67000000),
    )(sched, sh, adj, b1r, W2, b2r)
